# Initial kernel scaffold; baseline (speedup 1.0000x reference)
import jax
import jax.numpy as jnp
from jax.experimental import pallas as pl


def kernel(center_words, pos_words, neg_words, input_table, output_table):
    raise NotImplementedError("write your pallas kernel here")


# speedup vs baseline: 1.5856x; 1.5856x over previous
import functools

import jax
import jax.numpy as jnp
from jax import lax
from jax.experimental import pallas as pl
from jax.experimental.pallas import tpu as pltpu
from jax.experimental.pallas import tpu_sc as plsc

_L = 16


def _make_sc_scores(B, NEG, V, D):
    info = plsc.get_sparse_core_info()
    NC, NS = info.num_cores, info.num_subcores
    NW = NC * NS
    b_per_w = B // NW
    CHUNK = 128
    n_chunks = b_per_w // CHUNK
    mesh = plsc.VectorSubcoreMesh(core_axis_name="c", subcore_axis_name="s")

    @functools.partial(
        pl.kernel,
        out_type=jax.ShapeDtypeStruct((8, B), jnp.float32),
        mesh=mesh,
        compiler_params=pltpu.CompilerParams(
            use_tc_tiling_on_sc=False, needs_layout_passes=False),
        scratch_types=[
            pltpu.VMEM((CHUNK,), jnp.int32),
            pltpu.VMEM((CHUNK,), jnp.int32),
            pltpu.VMEM((NEG, CHUNK), jnp.int32),
            pltpu.VMEM((CHUNK, D), jnp.float32),
            pltpu.VMEM((CHUNK, D), jnp.float32),
            pltpu.VMEM((NEG, CHUNK, D), jnp.float32),
            pltpu.VMEM((8, CHUNK), jnp.float32),
            pltpu.SemaphoreType.DMA,
        ],
    )
    def sc_scores(cen_hbm, pos_hbm, negt_hbm, itab_hbm, otab_hbm, out_hbm,
                  idx_c, idx_p, idx_n, cen_v, pos_v, neg_v, sc_v, sem):
        wid = lax.axis_index("s") * NC + lax.axis_index("c")

        def chunk_body(ci, carry):
            base = wid * b_per_w + ci * CHUNK
            pltpu.sync_copy(cen_hbm.at[pl.ds(base, CHUNK)], idx_c)
            pltpu.sync_copy(pos_hbm.at[pl.ds(base, CHUNK)], idx_p)
            pltpu.sync_copy(negt_hbm.at[:, pl.ds(base, CHUNK)], idx_n)
            h_c = pltpu.async_copy(itab_hbm.at[idx_c], cen_v, sem)
            h_p = pltpu.async_copy(otab_hbm.at[idx_p], pos_v, sem)
            h_n = [pltpu.async_copy(otab_hbm.at[idx_n.at[k]], neg_v.at[k], sem)
                   for k in range(NEG)]
            h_c.wait()
            h_p.wait()
            for h in h_n:
                h.wait()

            def group_body(g, carry2):
                rows = g * _L + lax.broadcasted_iota(jnp.int32, (_L,), 0)

                def d_body(dd, accs):
                    acc_p, acc_n = accs
                    dsplat = jnp.full((_L,), dd, jnp.int32)
                    c = plsc.load_gather(cen_v, [rows, dsplat])
                    p = plsc.load_gather(pos_v, [rows, dsplat])
                    acc_p = acc_p + c * p
                    new_n = tuple(
                        acc_n[k] + c * plsc.load_gather(
                            neg_v,
                            [jnp.full((_L,), k, jnp.int32), rows, dsplat])
                        for k in range(NEG))
                    return acc_p, new_n

                z = jnp.zeros((_L,), jnp.float32)
                acc_p, acc_n = lax.fori_loop(0, D, d_body, (z, (z,) * NEG))
                sc_v[0, pl.ds(g * _L, _L)] = acc_p
                for k in range(NEG):
                    sc_v[1 + k, pl.ds(g * _L, _L)] = acc_n[k]
                sc_v[6, pl.ds(g * _L, _L)] = z
                sc_v[7, pl.ds(g * _L, _L)] = z
                return carry2

            lax.fori_loop(0, CHUNK // _L, group_body, 0)
            pltpu.sync_copy(sc_v, out_hbm.at[:, pl.ds(base, CHUNK)])
            return carry

        lax.fori_loop(0, n_chunks, chunk_body, 0)

    return sc_scores


def _loss_body(s_ref, o_ref):
    s = s_ref[...]
    rows = lax.broadcasted_iota(jnp.int32, s.shape, 0)
    x = jnp.where(rows == 0, s, -s)
    l = jnp.log(jax.nn.sigmoid(x) + 1e-10)
    l = jnp.where(rows < 6, l, 0.0)
    o_ref[0, 0] = -jnp.sum(l) / s.shape[1]


def kernel(center_words, pos_words, neg_words, input_table, output_table):
    B, = center_words.shape
    NEG = neg_words.shape[1]
    V, D = input_table.shape
    neg_t = jnp.transpose(neg_words)
    sc_scores = _make_sc_scores(B, NEG, V, D)
    scores = sc_scores(center_words, pos_words, neg_t,
                       input_table, output_table)
    loss = pl.pallas_call(
        _loss_body,
        out_shape=jax.ShapeDtypeStruct((1, 1), jnp.float32),
        out_specs=pl.BlockSpec(memory_space=pltpu.SMEM),
    )(scores)
    return jnp.reshape(loss, ())
